# baseline (speedup 1.0000x reference)
import functools

import jax
import jax.numpy as jnp
from jax import lax
from jax.experimental import pallas as pl
from jax.experimental.pallas import tpu as pltpu
from jax.experimental.pallas import tpu_sc as plsc

N = 50000
H = 256
G = 64
NC = 2
NS = 16
NW = NC * NS
RPW = 1600
TILE_A = 160
NWIN_A = RPW // TILE_A
TILE_C = 120
NWIN_C = (RPW + TILE_C - 1) // TILE_C
NSLOT = 3
HV = H // 16
PC = 2 * H + 128


def _sc_mesh():
    return plsc.VectorSubcoreMesh(
        core_axis_name="c", subcore_axis_name="s", num_cores=NC, num_subcores=NS
    )


def _sc_params():
    return pltpu.CompilerParams(needs_layout_passes=False)


def _al8(i):
    return pl.multiple_of(i, 8)


def _sload(ref, i):
    return ref[pl.ds(i, 16)][0]


def _worker_range():
    c = lax.axis_index("c")
    s = lax.axis_index("s")
    w = s * NC + c
    base = w * RPW
    cnt = jnp.minimum(RPW, N - base)
    return w, base, cnt


def _load_batch_slice(bat_hbm, bslice, base):
    s0b = _al8(jnp.minimum(base, N - RPW))
    pltpu.sync_copy(bat_hbm.at[pl.ds(s0b, RPW)], bslice.at[pl.ds(0, RPW)])
    return s0b


def _graph_span(bslice, s0b, base, cnt):
    g_first = _sload(bslice, base - s0b)
    g_last = _sload(bslice, base + cnt - 1 - s0b)
    return g_first, g_last - g_first + 1


def _run_end(bslice, s0b, g, pos, we):
    nblk = (we - pos + 15) // 16
    lanes = lax.iota(jnp.int32, 16)

    def scan_blk(b, first):
        q = pos + b * 16
        v = bslice[pl.ds(q - s0b, 16)]
        cand = jnp.where(v != g, q + lanes, N)
        return jnp.minimum(first, jnp.min(cand))

    first = lax.fori_loop(0, nblk, scan_blk, jnp.int32(N))
    return jnp.maximum(jnp.minimum(first, we), pos)


def _phase_a_body(x_hbm, bat_hbm, part_hbm, bslice, xbuf, acc, insem):
    w, base, cnt = _worker_range()

    def start_in(wi):
        ws = base + wi * TILE_A
        s0 = _al8(jnp.minimum(ws, N - TILE_A))
        pltpu.async_copy(
            x_hbm.at[pl.ds(s0, TILE_A), :], xbuf.at[wi % 2], insem.at[wi % 2]
        )

    start_in(0)
    s0b = _load_batch_slice(bat_hbm, bslice, base)
    g_first, gspan = _graph_span(bslice, s0b, base, cnt)

    def zero_body(g, carry):
        for j in range(PC // 16):
            acc[g, pl.ds(j * 16, 16)] = jnp.zeros((16,), jnp.float32)
        return carry

    lax.fori_loop(0, G, zero_body, 0)
    pos = base

    for wi in range(NWIN_A):
        p = wi % 2
        ws = base + wi * TILE_A
        we = jnp.minimum(ws + TILE_A, base + cnt)
        s0 = _al8(jnp.minimum(ws, N - TILE_A))
        pltpu.make_async_copy(
            x_hbm.at[pl.ds(s0, TILE_A), :], xbuf.at[p], insem.at[p]
        ).wait()
        if wi + 1 < NWIN_A:
            start_in(wi + 1)

        def seg_body(_, pos, we=we, s0=s0, p=p):
            g = jnp.clip(_sload(bslice, pos - s0b), 0, G - 1)
            hi = _run_end(bslice, s0b, g, pos, we)

            def row_pair(i, a2):
                sums = list(a2[:HV])
                sqs = list(a2[HV:])
                rb = pos - s0 + i * 2
                for j in range(HV):
                    v = xbuf[p, rb, pl.ds(j * 16, 16)]
                    u = xbuf[p, rb + 1, pl.ds(j * 16, 16)]
                    sums[j] = (sums[j] + v) + u
                    sqs[j] = (sqs[j] + v * v) + u * u
                return tuple(sums) + tuple(sqs)

            def row_body(r, a2):
                sums = list(a2[:HV])
                sqs = list(a2[HV:])
                for j in range(HV):
                    v = xbuf[p, r - s0, pl.ds(j * 16, 16)]
                    sums[j] = sums[j] + v
                    sqs[j] = sqs[j] + v * v
                return tuple(sums) + tuple(sqs)

            zeros = tuple(jnp.zeros((16,), jnp.float32) for _ in range(2 * HV))
            npair = (hi - pos) // 2
            accs = lax.fori_loop(0, npair, row_pair, zeros)
            accs = lax.fori_loop(pos + npair * 2, hi, row_body, accs)
            for j in range(2 * HV):
                plsc.addupdate(acc.at[g, pl.ds(j * 16, 16)], accs[j])
            cv = jnp.broadcast_to((hi - pos).astype(jnp.float32), (16,))
            plsc.addupdate(acc.at[g, pl.ds(2 * H, 16)], cv)
            return hi

        pos = lax.fori_loop(0, gspan, seg_body, pos)

    pltpu.sync_copy(acc, part_hbm.at[w])


def _phase_a(x, batch):
    f = pl.kernel(
        _phase_a_body,
        out_type=jax.ShapeDtypeStruct((NW, G, PC), jnp.float32),
        mesh=_sc_mesh(),
        compiler_params=_sc_params(),
        scratch_types=[
            pltpu.VMEM((RPW + 16,), jnp.int32),
            pltpu.VMEM((2, TILE_A, H), jnp.float32),
            pltpu.VMEM((G, PC), jnp.float32),
            pltpu.SemaphoreType.DMA((2,)),
        ],
    )
    return f(x, batch)


def _phase_b_body(part_ref, alpha_ref, weight_ref, bias_ref, st_ref):
    tot = part_ref[0]
    for i in range(1, NW):
        tot = tot + part_ref[i]
    sums = tot[:, :H]
    sqs = tot[:, H : 2 * H]
    counts = tot[:, 2 * H : 2 * H + 1]
    denom = jnp.maximum(counts, 1.0)
    a = alpha_ref[...]
    wgt = weight_ref[...]
    b = bias_ref[...]
    mean = sums / denom
    meansq = sqs / denom
    var = meansq - (2.0 * a - a * a) * mean * mean
    rstd = lax.rsqrt(jnp.maximum(var, 0.0) + 1e-6)
    s_tab = wgt * rstd
    t_tab = b - a * mean * s_tab
    st_ref[...] = jnp.concatenate([s_tab, t_tab], axis=1)


def _phase_b(part, alpha, weight, bias):
    return pl.pallas_call(
        _phase_b_body,
        out_shape=jax.ShapeDtypeStruct((G, 2 * H), jnp.float32),
    )(part, alpha, weight, bias)


def _phase_c_body(x_hbm, st_hbm, bat_hbm, y_hbm, bslice, stv, buf, insem, outsem):
    w, base, cnt = _worker_range()

    def start_in(wi):
        ws = base + wi * TILE_C
        s0 = _al8(jnp.minimum(ws, N - TILE_C))
        s = wi % NSLOT
        pltpu.async_copy(x_hbm.at[pl.ds(s0, TILE_C), :], buf.at[s], insem.at[s])

    start_in(0)
    pltpu.sync_copy(st_hbm, stv)
    s0b = _load_batch_slice(bat_hbm, bslice, base)
    g_first, gspan = _graph_span(bslice, s0b, base, cnt)

    def out_dma(wi, wait_only):
        ws = base + wi * TILE_C
        we = jnp.minimum(ws + TILE_C, base + cnt)
        s0 = _al8(jnp.minimum(ws, N - TILE_C))
        s = wi % NSLOT
        k = we - ws
        d = ws - s0

        @pl.when(k == TILE_C)
        def _():
            cp = pltpu.make_async_copy(
                buf.at[s], y_hbm.at[pl.ds(_al8(ws), TILE_C), :], outsem.at[s]
            )
            if wait_only:
                cp.wait()
            else:
                cp.start()

        @pl.when((k < TILE_C) & (k > 0))
        def _():
            for sz in (64, 32, 16, 8):
                pre = k & (~(2 * sz - 1))

                @pl.when((k & sz) != 0)
                def _():
                    cp = pltpu.make_async_copy(
                        buf.at[s, pl.ds(_al8(d + pre), sz), :],
                        y_hbm.at[pl.ds(_al8(ws + pre), sz), :],
                        outsem.at[s],
                    )
                    if wait_only:
                        cp.wait()
                    else:
                        cp.start()

    pos = base

    for wi in range(NWIN_C):
        s = wi % NSLOT
        ws = base + wi * TILE_C
        we = jnp.minimum(ws + TILE_C, base + cnt)
        s0 = _al8(jnp.minimum(ws, N - TILE_C))
        pltpu.make_async_copy(
            x_hbm.at[pl.ds(s0, TILE_C), :], buf.at[s], insem.at[s]
        ).wait()
        if wi + 1 < NWIN_C:
            if wi >= NSLOT - 1:
                out_dma(wi - (NSLOT - 1), wait_only=True)
            start_in(wi + 1)

        def seg_body(_, pos, we=we, s0=s0, s=s):
            g = jnp.clip(_sload(bslice, pos - s0b), 0, G - 1)
            hi = _run_end(bslice, s0b, g, pos, we)
            s_regs = [stv[g, pl.ds(j * 16, 16)] for j in range(HV)]
            t_regs = [stv[g, pl.ds(H + j * 16, 16)] for j in range(HV)]

            def row_body(r, c3):
                for j in range(HV):
                    v = buf[s, r - s0, pl.ds(j * 16, 16)]
                    buf[s, r - s0, pl.ds(j * 16, 16)] = v * s_regs[j] + t_regs[j]
                return c3

            lax.fori_loop(pos, hi, row_body, 0)
            return hi

        pos = lax.fori_loop(0, gspan, seg_body, pos)
        out_dma(wi, wait_only=False)

    for wi in range(max(NWIN_C - NSLOT, 0), NWIN_C):
        out_dma(wi, wait_only=True)


def _phase_c(x, st, batch):
    f = pl.kernel(
        _phase_c_body,
        out_type=jax.ShapeDtypeStruct((N, H), jnp.float32),
        mesh=_sc_mesh(),
        compiler_params=_sc_params(),
        scratch_types=[
            pltpu.VMEM((RPW + 16,), jnp.int32),
            pltpu.VMEM((G, 2 * H), jnp.float32),
            pltpu.VMEM((NSLOT, TILE_C, H), jnp.float32),
            pltpu.SemaphoreType.DMA((NSLOT,)),
            pltpu.SemaphoreType.DMA((NSLOT,)),
        ],
    )
    return f(x, st, batch)


@jax.jit
def kernel(x, batch, alpha, weight, bias):
    batch = batch.astype(jnp.int32)
    part = _phase_a(x, batch)
    st = _phase_b(part, alpha[None, :], weight[None, :], bias[None, :])
    return _phase_c(x, st, batch)


# speedup vs baseline: 1.0443x; 1.0443x over previous
import functools

import jax
import jax.numpy as jnp
from jax import lax
from jax.experimental import pallas as pl
from jax.experimental.pallas import tpu as pltpu
from jax.experimental.pallas import tpu_sc as plsc

N = 50000
H = 256
G = 64
NC = 2
NS = 16
NW = NC * NS
RPW = 1600
TILE_A = 160
NWIN_A = RPW // TILE_A
TILE_C = 120
NWIN_C = (RPW + TILE_C - 1) // TILE_C
NSLOT = 3
HV = H // 16
PC = 2 * H + 128


def _sc_mesh():
    return plsc.VectorSubcoreMesh(
        core_axis_name="c", subcore_axis_name="s", num_cores=NC, num_subcores=NS
    )


def _sc_params():
    return pltpu.CompilerParams(needs_layout_passes=False)


def _al8(i):
    return pl.multiple_of(i, 8)


def _sload(ref, i):
    return ref[pl.ds(i, 16)][0]


def _worker_range():
    c = lax.axis_index("c")
    s = lax.axis_index("s")
    w = s * NC + c
    base = w * RPW
    cnt = jnp.minimum(RPW, N - base)
    return w, base, cnt


def _load_batch_slice(bat_hbm, bslice, base):
    s0b = _al8(jnp.minimum(base, N - RPW))
    pltpu.sync_copy(bat_hbm.at[pl.ds(s0b, RPW)], bslice.at[pl.ds(0, RPW)])
    return s0b


def _graph_span(bslice, s0b, base, cnt):
    g_first = _sload(bslice, base - s0b)
    g_last = _sload(bslice, base + cnt - 1 - s0b)
    return g_first, g_last - g_first + 1


def _run_end(bslice, s0b, g, pos, we):
    nblk = (we - pos + 15) // 16
    lanes = lax.iota(jnp.int32, 16)

    def scan_blk(b, first):
        q = pos + b * 16
        v = bslice[pl.ds(q - s0b, 16)]
        cand = jnp.where(v != g, q + lanes, N)
        return jnp.minimum(first, jnp.min(cand))

    first = lax.fori_loop(0, nblk, scan_blk, jnp.int32(N))
    return jnp.maximum(jnp.minimum(first, we), pos)


def _phase_a_body(x_hbm, bat_hbm, part_hbm, bslice, xbuf, acc, insem):
    w, base, cnt = _worker_range()

    def start_in(wi):
        ws = base + wi * TILE_A
        s0 = _al8(jnp.minimum(ws, N - TILE_A))
        pltpu.async_copy(
            x_hbm.at[pl.ds(s0, TILE_A), :], xbuf.at[wi % 2], insem.at[wi % 2]
        )

    start_in(0)
    s0b = _load_batch_slice(bat_hbm, bslice, base)
    g_first, gspan = _graph_span(bslice, s0b, base, cnt)

    def zero_body(g, carry):
        for j in range(PC // 16):
            acc[g, pl.ds(j * 16, 16)] = jnp.zeros((16,), jnp.float32)
        return carry

    lax.fori_loop(0, G, zero_body, 0)
    pos = base

    for wi in range(NWIN_A):
        p = wi % 2
        ws = base + wi * TILE_A
        we = jnp.minimum(ws + TILE_A, base + cnt)
        s0 = _al8(jnp.minimum(ws, N - TILE_A))
        pltpu.make_async_copy(
            x_hbm.at[pl.ds(s0, TILE_A), :], xbuf.at[p], insem.at[p]
        ).wait()
        if wi + 1 < NWIN_A:
            start_in(wi + 1)

        def seg_body(_, pos, we=we, s0=s0, p=p):
            g = jnp.clip(_sload(bslice, pos - s0b), 0, G - 1)
            hi = _run_end(bslice, s0b, g, pos, we)

            def row_body(r, a2):
                sums = list(a2[:HV])
                sqs = list(a2[HV:])
                for j in range(HV):
                    v = xbuf[p, r - s0, pl.ds(j * 16, 16)]
                    sums[j] = sums[j] + v
                    sqs[j] = sqs[j] + v * v
                return tuple(sums) + tuple(sqs)

            zeros = tuple(jnp.zeros((16,), jnp.float32) for _ in range(2 * HV))
            accs = lax.fori_loop(pos, hi, row_body, zeros)
            for j in range(2 * HV):
                plsc.addupdate(acc.at[g, pl.ds(j * 16, 16)], accs[j])
            cv = jnp.broadcast_to((hi - pos).astype(jnp.float32), (16,))
            plsc.addupdate(acc.at[g, pl.ds(2 * H, 16)], cv)
            return hi

        pos = lax.fori_loop(0, gspan, seg_body, pos)

    pltpu.sync_copy(acc, part_hbm.at[w])


def _phase_a(x, batch):
    f = pl.kernel(
        _phase_a_body,
        out_type=jax.ShapeDtypeStruct((NW, G, PC), jnp.float32),
        mesh=_sc_mesh(),
        compiler_params=_sc_params(),
        scratch_types=[
            pltpu.VMEM((RPW + 16,), jnp.int32),
            pltpu.VMEM((2, TILE_A, H), jnp.float32),
            pltpu.VMEM((G, PC), jnp.float32),
            pltpu.SemaphoreType.DMA((2,)),
        ],
    )
    return f(x, batch)


def _phase_b_body(part_ref, alpha_ref, weight_ref, bias_ref, st_ref):
    tot = part_ref[0]
    for i in range(1, NW):
        tot = tot + part_ref[i]
    sums = tot[:, :H]
    sqs = tot[:, H : 2 * H]
    counts = tot[:, 2 * H : 2 * H + 1]
    denom = jnp.maximum(counts, 1.0)
    a = alpha_ref[...]
    wgt = weight_ref[...]
    b = bias_ref[...]
    mean = sums / denom
    meansq = sqs / denom
    var = meansq - (2.0 * a - a * a) * mean * mean
    rstd = lax.rsqrt(jnp.maximum(var, 0.0) + 1e-6)
    s_tab = wgt * rstd
    t_tab = b - a * mean * s_tab
    st_ref[...] = jnp.concatenate([s_tab, t_tab], axis=1)


def _phase_b(part, alpha, weight, bias):
    return pl.pallas_call(
        _phase_b_body,
        out_shape=jax.ShapeDtypeStruct((G, 2 * H), jnp.float32),
    )(part, alpha, weight, bias)


def _phase_c_body(x_hbm, st_hbm, bat_hbm, y_hbm, bslice, stv, buf, insem, outsem):
    w, base, cnt = _worker_range()

    def start_in(wi):
        ws = base + wi * TILE_C
        s0 = _al8(jnp.minimum(ws, N - TILE_C))
        s = wi % NSLOT
        pltpu.async_copy(x_hbm.at[pl.ds(s0, TILE_C), :], buf.at[s], insem.at[s])

    start_in(0)
    pltpu.sync_copy(st_hbm, stv)
    s0b = _load_batch_slice(bat_hbm, bslice, base)
    g_first, gspan = _graph_span(bslice, s0b, base, cnt)

    def out_dma(wi, wait_only):
        ws = base + wi * TILE_C
        we = jnp.minimum(ws + TILE_C, base + cnt)
        s0 = _al8(jnp.minimum(ws, N - TILE_C))
        s = wi % NSLOT
        k = we - ws
        d = ws - s0

        @pl.when(k == TILE_C)
        def _():
            cp = pltpu.make_async_copy(
                buf.at[s], y_hbm.at[pl.ds(_al8(ws), TILE_C), :], outsem.at[s]
            )
            if wait_only:
                cp.wait()
            else:
                cp.start()

        @pl.when((k < TILE_C) & (k > 0))
        def _():
            for sz in (64, 32, 16, 8):
                pre = k & (~(2 * sz - 1))

                @pl.when((k & sz) != 0)
                def _():
                    cp = pltpu.make_async_copy(
                        buf.at[s, pl.ds(_al8(d + pre), sz), :],
                        y_hbm.at[pl.ds(_al8(ws + pre), sz), :],
                        outsem.at[s],
                    )
                    if wait_only:
                        cp.wait()
                    else:
                        cp.start()

    pos = base

    for wi in range(NWIN_C):
        s = wi % NSLOT
        ws = base + wi * TILE_C
        we = jnp.minimum(ws + TILE_C, base + cnt)
        s0 = _al8(jnp.minimum(ws, N - TILE_C))
        pltpu.make_async_copy(
            x_hbm.at[pl.ds(s0, TILE_C), :], buf.at[s], insem.at[s]
        ).wait()
        if wi + 1 < NWIN_C:
            if wi >= NSLOT - 1:
                out_dma(wi - (NSLOT - 1), wait_only=True)
            start_in(wi + 1)

        def seg_body(_, pos, we=we, s0=s0, s=s):
            g = jnp.clip(_sload(bslice, pos - s0b), 0, G - 1)
            hi = _run_end(bslice, s0b, g, pos, we)
            s_regs = [stv[g, pl.ds(j * 16, 16)] for j in range(HV)]
            t_regs = [stv[g, pl.ds(H + j * 16, 16)] for j in range(HV)]

            def row_body(r, c3):
                for j in range(HV):
                    v = buf[s, r - s0, pl.ds(j * 16, 16)]
                    buf[s, r - s0, pl.ds(j * 16, 16)] = v * s_regs[j] + t_regs[j]
                return c3

            lax.fori_loop(pos, hi, row_body, 0)
            return hi

        pos = lax.fori_loop(0, gspan, seg_body, pos)
        out_dma(wi, wait_only=False)

    for wi in range(max(NWIN_C - NSLOT, 0), NWIN_C):
        out_dma(wi, wait_only=True)


def _phase_c(x, st, batch):
    f = pl.kernel(
        _phase_c_body,
        out_type=jax.ShapeDtypeStruct((N, H), jnp.float32),
        mesh=_sc_mesh(),
        compiler_params=_sc_params(),
        scratch_types=[
            pltpu.VMEM((RPW + 16,), jnp.int32),
            pltpu.VMEM((G, 2 * H), jnp.float32),
            pltpu.VMEM((NSLOT, TILE_C, H), jnp.float32),
            pltpu.SemaphoreType.DMA((NSLOT,)),
            pltpu.SemaphoreType.DMA((NSLOT,)),
        ],
    )
    return f(x, st, batch)


@jax.jit
def kernel(x, batch, alpha, weight, bias):
    batch = batch.astype(jnp.int32)
    part = _phase_a(x, batch)
    st = _phase_b(part, alpha[None, :], weight[None, :], bias[None, :])
    return _phase_c(x, st, batch)
